# baseline (speedup 1.0000x reference)
import functools
import math

import jax
import jax.numpy as jnp
from jax import lax
from jax.experimental import pallas as pl
from jax.experimental.pallas import tpu as pltpu
from jax.experimental.pallas import tpu_sc as plsc

_QLEN = 2048
_KLEN = 2048
_NUM_BUCKETS = 32
_N_HEADS = 16
_G1LEN = 4112
_BW = 4096
_THRESH = (10, 12, 14, 16, 20, 23, 27)
_NG = _QLEN // 2 // 8


def _sc_body(delta_hbm, table_hbm, out_hbm, tbl_v, dl_v, g1_v, bank_v, sem):
    cid = lax.axis_index("c")
    sid = lax.axis_index("s")
    wid = sid * 2 + cid
    h = wid // 2
    half = lax.rem(wid, 2)

    pltpu.sync_copy(table_hbm, tbl_v)
    pltpu.sync_copy(delta_hbm, dl_v)
    delta_vec = dl_v[...]
    hvec = jnp.full((16,), h, jnp.int32)

    def _build_chunk(ci, carry):
        y = lax.broadcasted_iota(jnp.int32, (16,), 0) + ci * 16
        rp = y - (8 + _QLEN) + delta_vec
        n = -rp
        ret = jnp.where(n < 0, _NUM_BUCKETS // 2, 0)
        na = jnp.abs(n)
        large = jnp.full((16,), 8, jnp.int32)
        for t in _THRESH:
            large = large + jnp.where(na >= t, 1, 0)
        bucket = ret + jnp.where(na < 8, na, large)
        g1_v[pl.ds(ci * 16, 16)] = plsc.load_gather(tbl_v, [bucket, hvec])
        return carry

    lax.fori_loop(0, _G1LEN // 16, _build_chunk, 0)

    def _bank_chunk(ci, carry):
        idx = lax.broadcasted_iota(jnp.int32, (16,), 0) + ci * 16
        for b in range(8):
            bank_v[pl.ds(b * _BW + ci * 16, 16)] = plsc.load_gather(
                g1_v, [idx + (8 - b)])
        return carry

    lax.fori_loop(0, _BW // 16, _bank_chunk, 0)

    base = half * (_QLEN // 2)

    def _copies(g):
        off = _QLEN - base - 8 * g
        row0 = (h * _QLEN + base + 8 * g) * _KLEN
        return [
            pltpu.make_async_copy(
                bank_v.at[pl.ds(b * _BW + off, _KLEN)],
                out_hbm.at[pl.ds(row0 + b * _KLEN, _KLEN)],
                sem,
            )
            for b in range(8)
        ]

    def _dma_group(g, carry):
        for c in _copies(g):
            c.start()

        @pl.when(g >= 2)
        def _retire():
            for c in _copies(g - 2):
                c.wait()

        return carry

    lax.fori_loop(0, _NG, _dma_group, 0)
    for gq in (_NG - 2, _NG - 1):
        for c in _copies(gq):
            c.wait()


@jax.jit
def _run(delta, table):
    mesh = plsc.VectorSubcoreMesh(core_axis_name="c", subcore_axis_name="s")
    fn = functools.partial(
        pl.kernel,
        mesh=mesh,
        compiler_params=pltpu.CompilerParams(needs_layout_passes=False),
        out_type=jax.ShapeDtypeStruct((_N_HEADS * _QLEN * _KLEN,), jnp.float32),
        scratch_types=[
            pltpu.VMEM((_NUM_BUCKETS, _N_HEADS), jnp.float32),
            pltpu.VMEM((16,), jnp.int32),
            pltpu.VMEM((_G1LEN,), jnp.float32),
            pltpu.VMEM((8 * _BW,), jnp.float32),
            pltpu.SemaphoreType.DMA,
        ],
    )(_sc_body)
    return fn(delta, table)


def kernel(qlen, klen, relative_attention_bias):
    qlen = jnp.asarray(qlen, jnp.int32)
    klen = jnp.asarray(klen, jnp.int32)
    delta = jnp.full((16,), (klen - _KLEN) - (qlen - _QLEN), jnp.int32)
    flat = _run(delta, relative_attention_bias)
    return flat.reshape(1, _N_HEADS, _QLEN, _KLEN)


# speedup vs baseline: 4.5774x; 4.5774x over previous
import functools
import math

import jax
import jax.numpy as jnp
from jax.experimental import pallas as pl
from jax.experimental.pallas import tpu as pltpu

_QLEN = 2048
_KLEN = 2048
_NUM_BUCKETS = 32
_N_HEADS = 16
_BQ = 512
_NQ = _QLEN // _BQ
_K = 8
_PAD = 4352


def _bias_kernel(delta_ref, tT_ref, out_ref, gb_ref, sem_ref):
    h = pl.program_id(0)
    qb = pl.program_id(1)
    p = h * _NQ + qb

    @pl.when(qb == 0)
    def _build_bank():
        d = jax.lax.broadcasted_iota(jnp.int32, (8, _PAD), 1)
        b = jax.lax.broadcasted_iota(jnp.int32, (8, _PAD), 0)
        rp = d - b - (_QLEN - delta_ref[0])
        n = -rp
        half = _NUM_BUCKETS // 2
        max_exact = half // 2
        ret = jnp.where(n < 0, half, 0).astype(jnp.int32)
        na = jnp.abs(n)
        is_small = na < max_exact
        naf = jnp.maximum(na, 1).astype(jnp.float32)
        t = (jnp.log(naf / max_exact) / math.log(32 / max_exact)
             * (half - max_exact)).astype(jnp.int32)
        val_large = jnp.minimum(max_exact + t, half - 1)
        bucket = ret + jnp.where(is_small, na, val_large)
        vals = jnp.zeros((8, _PAD), jnp.float32)
        for bkt in range(_NUM_BUCKETS):
            vals = jnp.where(bucket == bkt, tT_ref[0, 0, bkt], vals)
        hp = jax.lax.rem(h, 3)
        for a in range(_BQ // 8):
            gb_ref[hp, pl.ds(8 * a, 8), :] = (
                jnp.roll(vals, 8 * a, axis=1) if a else vals)

    def _copy(pi):
        hh = pi // _NQ
        qq = jax.lax.rem(pi, _NQ)
        return pltpu.make_async_copy(
            gb_ref.at[jax.lax.rem(hh, 3), :, pl.ds(_QLEN - qq * _BQ, _KLEN)],
            out_ref.at[0, hh, pl.ds(qq * _BQ, _BQ), :],
            sem_ref.at[jax.lax.rem(pi, _K)],
        )

    _copy(p).start()

    @pl.when(p >= _K)
    def _retire():
        _copy(p - _K).wait()

    last = _N_HEADS * _NQ - 1

    @pl.when(p == last)
    def _drain():
        for j in range(_K - 1, -1, -1):
            _copy(last - j).wait()


@jax.jit
def _run(delta, table_t):
    grid_spec = pltpu.PrefetchScalarGridSpec(
        num_scalar_prefetch=1,
        grid=(_N_HEADS, _NQ),
        in_specs=[pl.BlockSpec((1, 1, _NUM_BUCKETS), lambda h, q, *_: (h, 0, 0))],
        out_specs=pl.BlockSpec(memory_space=pl.ANY),
        scratch_shapes=[
            pltpu.VMEM((3, _BQ, _PAD), jnp.float32),
            pltpu.SemaphoreType.DMA((_K,)),
        ],
    )
    return pl.pallas_call(
        _bias_kernel,
        grid_spec=grid_spec,
        out_shape=jax.ShapeDtypeStruct((1, _N_HEADS, _QLEN, _KLEN), jnp.float32),
    )(delta, table_t)


def kernel(qlen, klen, relative_attention_bias):
    qlen = jnp.asarray(qlen, jnp.int32)
    klen = jnp.asarray(klen, jnp.int32)
    delta = ((klen - _KLEN) - (qlen - _QLEN)).reshape(1)
    table_t = relative_attention_bias.T.reshape(_N_HEADS, 1, _NUM_BUCKETS)
    return _run(delta, table_t)
